# Initial kernel scaffold; baseline (speedup 1.0000x reference)
import functools

import jax
import jax.numpy as jnp
from jax import lax
from jax.experimental import pallas as pl
from jax.experimental.pallas import tpu as pltpu
from jax.experimental.pallas import tpu_sc as plsc

N = 10000
E = 320000
D = 128
NUM_TYPES = 2

NW = 32
NPW = 313
LAST_ROWS = N - (NW - 1) * NPW
ACC_ROWS = NPW + 1
CHUNK = 1280
NCHUNK = E // CHUNK
B = 32
CAP = CHUNK + 2 * B
NEG_INF = float("-inf")

BLK = 512
GRID_I = (N + BLK - 1) // BLK


def _matmul_body(x_ref, w_ref, b_ref, y_ref):
    y_ref[0] = (
        jnp.dot(x_ref[...], w_ref[0], preferred_element_type=jnp.float32)
        + b_ref[...]
    )


def _compute_y(x, W, b):
    return pl.pallas_call(
        _matmul_body,
        grid=(NUM_TYPES, GRID_I),
        in_specs=[
            pl.BlockSpec((BLK, D), lambda t, i: (i, 0)),
            pl.BlockSpec((1, D, D), lambda t, i: (t, 0, 0)),
            pl.BlockSpec((1, D), lambda t, i: (t, 0)),
        ],
        out_specs=pl.BlockSpec((1, BLK, D), lambda t, i: (t, i, 0)),
        out_shape=jax.ShapeDtypeStruct((NUM_TYPES, N, D), jnp.float32),
    )(x, W, b)


_MESH = plsc.VectorSubcoreMesh(core_axis_name="c", subcore_axis_name="s")


@functools.partial(
    pl.kernel,
    out_type=jax.ShapeDtypeStruct((N * D,), jnp.float32),
    mesh=_MESH,
    scratch_types=[
        pltpu.VMEM((CHUNK,), jnp.int32),
        pltpu.VMEM((CHUNK,), jnp.int32),
        pltpu.VMEM((CAP,), jnp.int32),
        pltpu.VMEM((CAP,), jnp.int32),
        pltpu.VMEM((B, D), jnp.float32),
        pltpu.VMEM((ACC_ROWS * D,), jnp.float32),
        pltpu.SemaphoreType.DMA,
    ],
)
def _sc_gather_max(y_ref, dst_ref, row_ref, out_ref,
                   dbuf, rbuf, pend_off, pend_row, msg, acc, sem_g):
    c = lax.axis_index("c")
    s = lax.axis_index("s")
    wid = c * 16 + s
    base = wid * NPW
    n_rows = jnp.where(wid == NW - 1, LAST_ROWS, NPW)

    minus_inf = jnp.full((16,), NEG_INF, jnp.float32)
    full_mask = jnp.full((16,), True, jnp.bool_)

    def init_body(i, carry):
        acc[pl.ds(i * 16, 16)] = minus_inf
        return carry

    lax.fori_loop(0, ACC_ROWS * D // 16, init_body, 0)

    def process_batch(p):
        cp = pltpu.async_copy(y_ref.at[pend_row.at[pl.ds(p, B)]], msg, sem_g)
        cp.wait()

        def upd(i, carry):
            off = pend_off[p + i]
            a0 = off * D
            for j in range(D // 16):
                mv = msg[i, pl.ds(j * 16, 16)]
                av = acc[pl.ds(a0 + j * 16, 16)]
                acc[pl.ds(a0 + j * 16, 16)] = jnp.maximum(av, mv)
            return carry

        lax.fori_loop(0, B, upd, 0)

    def chunk_body(ci, k):
        pltpu.sync_copy(dst_ref.at[pl.ds(ci * CHUNK, CHUNK)], dbuf)
        pltpu.sync_copy(row_ref.at[pl.ds(ci * CHUNK, CHUNK)], rbuf)

        def scan_body(v, k):
            d = dbuf[pl.ds(v * 16, 16)]
            off = d - base
            m = (off >= 0) & (off < n_rows)
            cnt = jnp.max(plsc.all_reduce_population_count(m))

            @pl.when(cnt > 0)
            def _():
                plsc.store_compressed(pend_off.at[pl.ds(k, 16)], off, m)
                r = rbuf[pl.ds(v * 16, 16)]
                plsc.store_compressed(pend_row.at[pl.ds(k, 16)], r, m)

            return k + cnt

        k = lax.fori_loop(0, CHUNK // 16, scan_body, k)

        nb = k // B

        def pb(ib, carry):
            process_batch(ib * B)
            return carry

        lax.fori_loop(0, nb, pb, 0)

        src = nb * B
        for t in range(B // 16):
            v_off = pend_off[pl.ds(src + t * 16, 16)]
            pend_off[pl.ds(t * 16, 16)] = v_off
            v_row = pend_row[pl.ds(src + t * 16, 16)]
            pend_row[pl.ds(t * 16, 16)] = v_row
        return k - src

    k = lax.fori_loop(0, NCHUNK, chunk_body, jnp.int32(0))

    @pl.when(k > 0)
    def _():
        dump = jnp.full((16,), NPW, jnp.int32)
        zero16 = jnp.zeros((16,), jnp.int32)
        plsc.store_compressed(pend_off.at[pl.ds(k, 16)], dump, full_mask)
        plsc.store_compressed(pend_off.at[pl.ds(k + 16, 16)], dump, full_mask)
        plsc.store_compressed(pend_row.at[pl.ds(k, 16)], zero16, full_mask)
        plsc.store_compressed(pend_row.at[pl.ds(k + 16, 16)], zero16, full_mask)
        process_batch(0)

    def fix_body(i, carry):
        a = acc[pl.ds(i * 16, 16)]
        acc[pl.ds(i * 16, 16)] = jnp.where(a == NEG_INF, 0.0, a)
        return carry

    lax.fori_loop(0, ACC_ROWS * D // 16, fix_body, 0)

    @pl.when(wid < NW - 1)
    def _():
        pltpu.sync_copy(acc.at[pl.ds(0, NPW * D)],
                        out_ref.at[pl.ds(base * D, NPW * D)])

    @pl.when(wid == NW - 1)
    def _():
        pltpu.sync_copy(acc.at[pl.ds(0, LAST_ROWS * D)],
                        out_ref.at[pl.ds(base * D, LAST_ROWS * D)])


def kernel(x, edge_index, edge_attr, W, b):
    y = _compute_y(x, W, b)
    yflat = y.reshape(NUM_TYPES * N, D)
    src = edge_index[0]
    dst = edge_index[1]
    rowidx = edge_attr * N + src
    outflat = _sc_gather_max(yflat, dst, rowidx)
    return outflat.reshape(N, D)


# speedup vs baseline: 1.8308x; 1.8308x over previous
import functools

import jax
import jax.numpy as jnp
from jax import lax
from jax.experimental import pallas as pl
from jax.experimental.pallas import tpu as pltpu
from jax.experimental.pallas import tpu_sc as plsc

N = 10000
E = 320000
D = 128
NUM_TYPES = 2

NW = 32
NPW = 313
LAST_ROWS = N - (NW - 1) * NPW
ACC_ROWS = NPW + 1
CHUNK = 1280
NCHUNK = E // CHUNK
B = 32
CAP = CHUNK + 2 * B
NEG_INF = float("-inf")

BLK = 512
GRID_I = (N + BLK - 1) // BLK


def _matmul_body(x_ref, w_ref, b_ref, y_ref):
    y_ref[0] = (
        jnp.dot(x_ref[...], w_ref[0], preferred_element_type=jnp.float32)
        + b_ref[0]
    )


def _compute_y(x, W, b):
    return pl.pallas_call(
        _matmul_body,
        grid=(NUM_TYPES, GRID_I),
        in_specs=[
            pl.BlockSpec((BLK, D), lambda t, i: (i, 0)),
            pl.BlockSpec((1, D, D), lambda t, i: (t, 0, 0)),
            pl.BlockSpec((1, 1, D), lambda t, i: (t, 0, 0)),
        ],
        out_specs=pl.BlockSpec((1, BLK, D), lambda t, i: (t, i, 0)),
        out_shape=jax.ShapeDtypeStruct((NUM_TYPES, N, D), jnp.float32),
    )(x, W, b.reshape(NUM_TYPES, 1, D))


_MESH = plsc.VectorSubcoreMesh(core_axis_name="c", subcore_axis_name="s")


@functools.partial(
    pl.kernel,
    out_type=jax.ShapeDtypeStruct((N * D,), jnp.float32),
    mesh=_MESH,
    scratch_types=[
        pltpu.VMEM((CHUNK,), jnp.int32),
        pltpu.VMEM((CHUNK,), jnp.int32),
        pltpu.VMEM((CAP,), jnp.int32),
        pltpu.VMEM((CAP,), jnp.int32),
        pltpu.VMEM((B, D), jnp.float32),
        pltpu.VMEM((ACC_ROWS * D,), jnp.float32),
        pltpu.SemaphoreType.DMA,
    ],
    compiler_params=pltpu.CompilerParams(needs_layout_passes=False),
)
def _sc_gather_max(y_ref, dst_ref, row_ref, out_ref,
                   dbuf, rbuf, pend_off, pend_row, msg, acc, sem_g):
    c = lax.axis_index("c")
    s = lax.axis_index("s")
    wid = c * 16 + s
    base = wid * NPW
    n_rows = jnp.where(wid == NW - 1, LAST_ROWS, NPW)

    minus_inf = jnp.full((16,), NEG_INF, jnp.float32)
    full_mask = jnp.full((16,), True, jnp.bool_)

    def init_body(i, carry):
        acc[pl.ds(i * 16, 16)] = minus_inf
        return carry

    lax.fori_loop(0, ACC_ROWS * D // 16, init_body, 0)

    def process_batch(p):
        cp = pltpu.async_copy(y_ref.at[pend_row.at[pl.ds(p, B)]], msg, sem_g)
        cp.wait()

        def upd_group(g, carry):
            off16 = pend_off[pl.ds(p + g * 16, 16)]
            for i in range(16):
                a0 = off16[i] * D
                r = g * 16 + i
                for j in range(D // 16):
                    mv = msg[r, pl.ds(j * 16, 16)]
                    av = acc[pl.ds(a0 + j * 16, 16)]
                    acc[pl.ds(a0 + j * 16, 16)] = jnp.maximum(av, mv)
            return carry

        lax.fori_loop(0, B // 16, upd_group, 0)

    def chunk_body(ci, k):
        pltpu.sync_copy(dst_ref.at[pl.ds(ci * CHUNK, CHUNK)], dbuf)
        pltpu.sync_copy(row_ref.at[pl.ds(ci * CHUNK, CHUNK)], rbuf)

        def scan_body(v, k):
            d = dbuf[pl.ds(v * 16, 16)]
            off = d - base
            m = (off >= 0) & (off < n_rows)
            cnt = jnp.sum(m.astype(jnp.int32))

            @pl.when(cnt > 0)
            def _():
                plsc.store_compressed(pend_off.at[pl.ds(k, 16)], off, mask=m)
                r = rbuf[pl.ds(v * 16, 16)]
                plsc.store_compressed(pend_row.at[pl.ds(k, 16)], r, mask=m)

            return k + cnt

        k = lax.fori_loop(0, CHUNK // 16, scan_body, k)

        nb = k // B

        def pb(ib, carry):
            process_batch(ib * B)
            return carry

        lax.fori_loop(0, nb, pb, 0)

        src = nb * B
        for t in range(B // 16):
            v_off = pend_off[pl.ds(src + t * 16, 16)]
            pend_off[pl.ds(t * 16, 16)] = v_off
            v_row = pend_row[pl.ds(src + t * 16, 16)]
            pend_row[pl.ds(t * 16, 16)] = v_row
        return k - src

    k = lax.fori_loop(0, NCHUNK, chunk_body, jnp.int32(0))

    @pl.when(k > 0)
    def _():
        dump = jnp.full((16,), NPW, jnp.int32)
        zero16 = jnp.zeros((16,), jnp.int32)
        plsc.store_compressed(pend_off.at[pl.ds(k, 16)], dump, mask=full_mask)
        plsc.store_compressed(pend_off.at[pl.ds(k + 16, 16)], dump, mask=full_mask)
        plsc.store_compressed(pend_row.at[pl.ds(k, 16)], zero16, mask=full_mask)
        plsc.store_compressed(pend_row.at[pl.ds(k + 16, 16)], zero16, mask=full_mask)
        process_batch(0)

    def fix_body(i, carry):
        a = acc[pl.ds(i * 16, 16)]
        acc[pl.ds(i * 16, 16)] = jnp.where(a == NEG_INF, 0.0, a)
        return carry

    lax.fori_loop(0, ACC_ROWS * D // 16, fix_body, 0)

    @pl.when(wid < NW - 1)
    def _():
        pltpu.sync_copy(acc.at[pl.ds(0, NPW * D)],
                        out_ref.at[pl.ds(base * D, NPW * D)])

    @pl.when(wid == NW - 1)
    def _():
        pltpu.sync_copy(acc.at[pl.ds(0, LAST_ROWS * D)],
                        out_ref.at[pl.ds(base * D, LAST_ROWS * D)])


def kernel(x, edge_index, edge_attr, W, b):
    y = _compute_y(x, W, b)
    yflat = y.reshape(NUM_TYPES * N, D)
    src = edge_index[0]
    dst = edge_index[1]
    rowidx = edge_attr * N + src
    outflat = _sc_gather_max(yflat, dst, rowidx)
    return outflat.reshape(N, D)


# speedup vs baseline: 4.2590x; 2.3264x over previous
import functools

import jax
import jax.numpy as jnp
from jax import lax
from jax.experimental import pallas as pl
from jax.experimental.pallas import tpu as pltpu
from jax.experimental.pallas import tpu_sc as plsc

N = 10000
E = 320000
D = 128
NUM_TYPES = 2

NW = 32
NPW = 313
LAST_ROWS = N - (NW - 1) * NPW
ACC_ROWS = NPW + 1
CHUNK = 1280
NCHUNK = E // CHUNK
B = 32
THRESH = 2048
CAP = THRESH + CHUNK + 2 * B
NEG_INF = float("-inf")

BLK = 512
GRID_I = (N + BLK - 1) // BLK


def _matmul_body(x_ref, w_ref, b_ref, y_ref):
    y_ref[0] = (
        jnp.dot(x_ref[...], w_ref[0], preferred_element_type=jnp.float32)
        + b_ref[0]
    )


def _compute_y(x, W, b):
    return pl.pallas_call(
        _matmul_body,
        grid=(NUM_TYPES, GRID_I),
        in_specs=[
            pl.BlockSpec((BLK, D), lambda t, i: (i, 0)),
            pl.BlockSpec((1, D, D), lambda t, i: (t, 0, 0)),
            pl.BlockSpec((1, 1, D), lambda t, i: (t, 0, 0)),
        ],
        out_specs=pl.BlockSpec((1, BLK, D), lambda t, i: (t, i, 0)),
        out_shape=jax.ShapeDtypeStruct((NUM_TYPES, N, D), jnp.float32),
    )(x, W, b.reshape(NUM_TYPES, 1, D))


_MESH = plsc.VectorSubcoreMesh(core_axis_name="c", subcore_axis_name="s")


@functools.partial(
    pl.kernel,
    out_type=jax.ShapeDtypeStruct((N * D,), jnp.float32),
    mesh=_MESH,
    scratch_types=[
        pltpu.VMEM((CHUNK,), jnp.int32),
        pltpu.VMEM((CHUNK,), jnp.int32),
        pltpu.VMEM((CHUNK,), jnp.int32),
        pltpu.VMEM((CHUNK,), jnp.int32),
        pltpu.VMEM((CAP,), jnp.int32),
        pltpu.VMEM((CAP,), jnp.int32),
        pltpu.VMEM((B, D), jnp.float32),
        pltpu.VMEM((B, D), jnp.float32),
        pltpu.VMEM((ACC_ROWS * D,), jnp.float32),
        pltpu.SemaphoreType.DMA,
        pltpu.SemaphoreType.DMA,
        pltpu.SemaphoreType.DMA,
        pltpu.SemaphoreType.DMA,
        pltpu.SemaphoreType.DMA,
        pltpu.SemaphoreType.DMA,
    ],
    compiler_params=pltpu.CompilerParams(needs_layout_passes=False),
)
def _sc_gather_max(y_ref, dst_ref, row_ref, out_ref,
                   dst0, dst1, row0, row1, pend_off, pend_row,
                   msg0, msg1, acc, sd0, sd1, sr0, sr1, sg0, sg1):
    c = lax.axis_index("c")
    s = lax.axis_index("s")
    wid = c * 16 + s
    base = wid * NPW
    n_rows = jnp.where(wid == NW - 1, LAST_ROWS, NPW)

    minus_inf = jnp.full((16,), NEG_INF, jnp.float32)
    full_mask = jnp.full((16,), True, jnp.bool_)

    def init_body(i, carry):
        for u in range(4):
            acc[pl.ds((i * 4 + u) * 16, 16)] = minus_inf
        return carry

    lax.fori_loop(0, ACC_ROWS * D // 64, init_body, 0)

    def issue_chunk(ci, dbuf, rbuf, sd, sr):
        pltpu.async_copy(dst_ref.at[pl.ds(ci * CHUNK, CHUNK)], dbuf, sd)
        pltpu.async_copy(row_ref.at[pl.ds(ci * CHUNK, CHUNK)], rbuf, sr)

    def wait_chunk(dbuf, rbuf, sd, sr):
        pltpu.make_async_copy(dst_ref.at[pl.ds(0, CHUNK)], dbuf, sd).wait()
        pltpu.make_async_copy(row_ref.at[pl.ds(0, CHUNK)], rbuf, sr).wait()

    def gi(p, mref, sref):
        pltpu.async_copy(y_ref.at[pend_row.at[pl.ds(p, B)]], mref, sref)

    def gw(mref, sref):
        pltpu.make_async_copy(y_ref.at[pend_row.at[pl.ds(0, B)]], mref, sref).wait()

    def upd_batch(mref, p):
        def g_body(g, carry):
            off16 = pend_off[pl.ds(p + g * 16, 16)]
            for i in range(16):
                a0 = off16[i] * D
                r = g * 16 + i
                for j in range(D // 16):
                    mv = mref[r, pl.ds(j * 16, 16)]
                    av = acc[pl.ds(a0 + j * 16, 16)]
                    acc[pl.ds(a0 + j * 16, 16)] = jnp.maximum(av, mv)
            return carry

        lax.fori_loop(0, B // 16, g_body, 0)

    def run_batches(nb):
        gi(0, msg0, sg0)

        @pl.when(nb > 1)
        def _():
            gi(B, msg1, sg1)

        def body(t, carry):
            b0 = 2 * t
            gw(msg0, sg0)
            upd_batch(msg0, b0 * B)

            @pl.when(b0 + 2 < nb)
            def _():
                gi((b0 + 2) * B, msg0, sg0)

            @pl.when(b0 + 1 < nb)
            def _():
                gw(msg1, sg1)
                upd_batch(msg1, (b0 + 1) * B)

                @pl.when(b0 + 3 < nb)
                def _():
                    gi((b0 + 3) * B, msg1, sg1)

            return carry

        lax.fori_loop(0, (nb + 1) // 2, body, 0)

    def process_pending(k, thresh):
        def do():
            nb = k // B
            run_batches(nb)
            src = nb * B
            for t in range(B // 16):
                v_off = pend_off[pl.ds(src + t * 16, 16)]
                pend_off[pl.ds(t * 16, 16)] = v_off
                v_row = pend_row[pl.ds(src + t * 16, 16)]
                pend_row[pl.ds(t * 16, 16)] = v_row
            return k - src

        return lax.cond(k >= thresh, do, lambda: k)

    def scan_chunk(dbuf, rbuf, k):
        def scan_body(v, k):
            offs = []
            masks = []
            cnts = []
            for u in range(4):
                d = dbuf[pl.ds((v * 4 + u) * 16, 16)]
                o = d - base
                m = (o >= 0) & (o < n_rows)
                offs.append(o)
                masks.append(m)
                cnts.append(jnp.sum(m.astype(jnp.int32)))
            kpos = [k]
            for u in range(3):
                kpos.append(kpos[-1] + cnts[u])
            for u in range(4):
                @pl.when(cnts[u] > 0)
                def _(u=u):
                    plsc.store_compressed(
                        pend_off.at[pl.ds(kpos[u], 16)], offs[u], mask=masks[u])
                    r = rbuf[pl.ds((v * 4 + u) * 16, 16)]
                    plsc.store_compressed(
                        pend_row.at[pl.ds(kpos[u], 16)], r, mask=masks[u])
            return kpos[3] + cnts[3]

        return lax.fori_loop(0, CHUNK // 64, scan_body, k)

    issue_chunk(0, dst0, row0, sd0, sr0)

    def pair_body(t, k):
        c0 = 2 * t
        wait_chunk(dst0, row0, sd0, sr0)
        issue_chunk(c0 + 1, dst1, row1, sd1, sr1)
        k = scan_chunk(dst0, row0, k)
        k = process_pending(k, THRESH)
        wait_chunk(dst1, row1, sd1, sr1)

        @pl.when(c0 + 2 < NCHUNK)
        def _():
            issue_chunk(c0 + 2, dst0, row0, sd0, sr0)

        k = scan_chunk(dst1, row1, k)
        k = process_pending(k, THRESH)
        return k

    k = lax.fori_loop(0, NCHUNK // 2, pair_body, jnp.int32(0))

    k = process_pending(k, B)

    @pl.when(k > 0)
    def _():
        dump = jnp.full((16,), NPW, jnp.int32)
        zero16 = jnp.zeros((16,), jnp.int32)
        plsc.store_compressed(pend_off.at[pl.ds(k, 16)], dump, mask=full_mask)
        plsc.store_compressed(pend_off.at[pl.ds(k + 16, 16)], dump, mask=full_mask)
        plsc.store_compressed(pend_row.at[pl.ds(k, 16)], zero16, mask=full_mask)
        plsc.store_compressed(pend_row.at[pl.ds(k + 16, 16)], zero16, mask=full_mask)
        run_batches(1)

    def fix_body(i, carry):
        for u in range(4):
            a = acc[pl.ds((i * 4 + u) * 16, 16)]
            acc[pl.ds((i * 4 + u) * 16, 16)] = jnp.where(a == NEG_INF, 0.0, a)
        return carry

    lax.fori_loop(0, ACC_ROWS * D // 64, fix_body, 0)

    @pl.when(wid < NW - 1)
    def _():
        pltpu.sync_copy(acc.at[pl.ds(0, NPW * D)],
                        out_ref.at[pl.ds(base * D, NPW * D)])

    @pl.when(wid == NW - 1)
    def _():
        pltpu.sync_copy(acc.at[pl.ds(0, LAST_ROWS * D)],
                        out_ref.at[pl.ds(base * D, LAST_ROWS * D)])


def kernel(x, edge_index, edge_attr, W, b):
    y = _compute_y(x, W, b)
    yflat = y.reshape(NUM_TYPES * N, D)
    src = edge_index[0]
    dst = edge_index[1]
    rowidx = edge_attr * N + src
    outflat = _sc_gather_max(yflat, dst, rowidx)
    return outflat.reshape(N, D)


# speedup vs baseline: 4.3401x; 1.0190x over previous
import functools

import jax
import jax.numpy as jnp
from jax import lax
from jax.experimental import pallas as pl
from jax.experimental.pallas import tpu as pltpu
from jax.experimental.pallas import tpu_sc as plsc

N = 10000
E = 320000
D = 128
NUM_TYPES = 2

NW = 32
NPW = 313
LAST_ROWS = N - (NW - 1) * NPW
ACC_ROWS = NPW + 1
CHUNK = 1280
NCHUNK = E // CHUNK
B = 32
THRESH = 2048
CAP = THRESH + CHUNK + 2 * B
NEG_INF = float("-inf")

BLK = 512
GRID_I = (N + BLK - 1) // BLK


def _matmul_body(x_ref, w_ref, b_ref, y_ref):
    y_ref[0] = (
        jnp.dot(x_ref[...], w_ref[0], preferred_element_type=jnp.float32)
        + b_ref[0]
    )


def _compute_y(x, W, b):
    return pl.pallas_call(
        _matmul_body,
        grid=(NUM_TYPES, GRID_I),
        in_specs=[
            pl.BlockSpec((BLK, D), lambda t, i: (i, 0)),
            pl.BlockSpec((1, D, D), lambda t, i: (t, 0, 0)),
            pl.BlockSpec((1, 1, D), lambda t, i: (t, 0, 0)),
        ],
        out_specs=pl.BlockSpec((1, BLK, D), lambda t, i: (t, i, 0)),
        out_shape=jax.ShapeDtypeStruct((NUM_TYPES, N, D), jnp.float32),
    )(x, W, b.reshape(NUM_TYPES, 1, D))


_MESH = plsc.VectorSubcoreMesh(core_axis_name="c", subcore_axis_name="s")


@functools.partial(
    pl.kernel,
    out_type=jax.ShapeDtypeStruct((N * D,), jnp.float32),
    mesh=_MESH,
    scratch_types=[
        pltpu.VMEM((CHUNK,), jnp.int32),
        pltpu.VMEM((CHUNK,), jnp.int32),
        pltpu.VMEM((CHUNK,), jnp.int32),
        pltpu.VMEM((CHUNK,), jnp.int32),
        pltpu.VMEM((CAP,), jnp.int32),
        pltpu.VMEM((CAP,), jnp.int32),
        pltpu.VMEM((B, D), jnp.float32),
        pltpu.VMEM((B, D), jnp.float32),
        pltpu.VMEM((ACC_ROWS * D,), jnp.float32),
        pltpu.VMEM((ACC_ROWS * D,), jnp.float32),
        pltpu.SemaphoreType.DMA,
        pltpu.SemaphoreType.DMA,
        pltpu.SemaphoreType.DMA,
        pltpu.SemaphoreType.DMA,
        pltpu.SemaphoreType.DMA,
        pltpu.SemaphoreType.DMA,
    ],
    compiler_params=pltpu.CompilerParams(needs_layout_passes=False),
)
def _sc_gather_max(y_ref, dst_ref, row_ref, out_ref,
                   dst0, dst1, row0, row1, pend_off, pend_row,
                   msg0, msg1, acc, acc2, sd0, sd1, sr0, sr1, sg0, sg1):
    c = lax.axis_index("c")
    s = lax.axis_index("s")
    wid = c * 16 + s
    base = wid * NPW
    n_rows = jnp.where(wid == NW - 1, LAST_ROWS, NPW)

    minus_inf = jnp.full((16,), NEG_INF, jnp.float32)
    full_mask = jnp.full((16,), True, jnp.bool_)

    def init_body(i, carry):
        for u in range(4):
            acc[pl.ds((i * 4 + u) * 16, 16)] = minus_inf
            acc2[pl.ds((i * 4 + u) * 16, 16)] = minus_inf
        return carry

    lax.fori_loop(0, ACC_ROWS * D // 64, init_body, 0)

    def issue_chunk(ci, dbuf, rbuf, sd, sr):
        pltpu.async_copy(dst_ref.at[pl.ds(ci * CHUNK, CHUNK)], dbuf, sd)
        pltpu.async_copy(row_ref.at[pl.ds(ci * CHUNK, CHUNK)], rbuf, sr)

    def wait_chunk(dbuf, rbuf, sd, sr):
        pltpu.make_async_copy(dst_ref.at[pl.ds(0, CHUNK)], dbuf, sd).wait()
        pltpu.make_async_copy(row_ref.at[pl.ds(0, CHUNK)], rbuf, sr).wait()

    def gi(p, mref, sref):
        pltpu.async_copy(y_ref.at[pend_row.at[pl.ds(p, B)]], mref, sref)

    def gw(mref, sref):
        pltpu.make_async_copy(y_ref.at[pend_row.at[pl.ds(0, B)]], mref, sref).wait()

    def upd_batch(mref, p):
        def g_body(g, carry):
            off16 = pend_off[pl.ds(p + g * 16, 16)]
            for i in range(16):
                a0 = off16[i] * D
                r = g * 16 + i
                tgt = acc if i % 2 == 0 else acc2
                for j in range(D // 16):
                    mv = mref[r, pl.ds(j * 16, 16)]
                    av = tgt[pl.ds(a0 + j * 16, 16)]
                    tgt[pl.ds(a0 + j * 16, 16)] = jnp.maximum(av, mv)
            return carry

        lax.fori_loop(0, B // 16, g_body, 0)

    def run_batches(nb):
        gi(0, msg0, sg0)

        @pl.when(nb > 1)
        def _():
            gi(B, msg1, sg1)

        def body(t, carry):
            b0 = 2 * t
            gw(msg0, sg0)
            upd_batch(msg0, b0 * B)

            @pl.when(b0 + 2 < nb)
            def _():
                gi((b0 + 2) * B, msg0, sg0)

            @pl.when(b0 + 1 < nb)
            def _():
                gw(msg1, sg1)
                upd_batch(msg1, (b0 + 1) * B)

                @pl.when(b0 + 3 < nb)
                def _():
                    gi((b0 + 3) * B, msg1, sg1)

            return carry

        lax.fori_loop(0, (nb + 1) // 2, body, 0)

    def process_pending(k, thresh):
        def do():
            nb = k // B
            run_batches(nb)
            src = nb * B
            for t in range(B // 16):
                v_off = pend_off[pl.ds(src + t * 16, 16)]
                pend_off[pl.ds(t * 16, 16)] = v_off
                v_row = pend_row[pl.ds(src + t * 16, 16)]
                pend_row[pl.ds(t * 16, 16)] = v_row
            return k - src

        return lax.cond(k >= thresh, do, lambda: k)

    def scan_chunk(dbuf, rbuf, k):
        def scan_body(v, k):
            offs = []
            masks = []
            cnts = []
            for u in range(4):
                d = dbuf[pl.ds((v * 4 + u) * 16, 16)]
                o = d - base
                m = (o >= 0) & (o < n_rows)
                offs.append(o)
                masks.append(m)
                cnts.append(plsc.all_reduce_population_count(m)[0])
            kpos = [k]
            for u in range(3):
                kpos.append(kpos[-1] + cnts[u])
            for u in range(4):
                @pl.when(cnts[u] > 0)
                def _(u=u):
                    plsc.store_compressed(
                        pend_off.at[pl.ds(kpos[u], 16)], offs[u], mask=masks[u])
                    r = rbuf[pl.ds((v * 4 + u) * 16, 16)]
                    plsc.store_compressed(
                        pend_row.at[pl.ds(kpos[u], 16)], r, mask=masks[u])
            return kpos[3] + cnts[3]

        return lax.fori_loop(0, CHUNK // 64, scan_body, k)

    issue_chunk(0, dst0, row0, sd0, sr0)

    def pair_body(t, k):
        c0 = 2 * t
        wait_chunk(dst0, row0, sd0, sr0)
        issue_chunk(c0 + 1, dst1, row1, sd1, sr1)
        k = scan_chunk(dst0, row0, k)
        k = process_pending(k, THRESH)
        wait_chunk(dst1, row1, sd1, sr1)

        @pl.when(c0 + 2 < NCHUNK)
        def _():
            issue_chunk(c0 + 2, dst0, row0, sd0, sr0)

        k = scan_chunk(dst1, row1, k)
        k = process_pending(k, THRESH)
        return k

    k = lax.fori_loop(0, NCHUNK // 2, pair_body, jnp.int32(0))

    k = process_pending(k, B)

    @pl.when(k > 0)
    def _():
        dump = jnp.full((16,), NPW, jnp.int32)
        zero16 = jnp.zeros((16,), jnp.int32)
        plsc.store_compressed(pend_off.at[pl.ds(k, 16)], dump, mask=full_mask)
        plsc.store_compressed(pend_off.at[pl.ds(k + 16, 16)], dump, mask=full_mask)
        plsc.store_compressed(pend_row.at[pl.ds(k, 16)], zero16, mask=full_mask)
        plsc.store_compressed(pend_row.at[pl.ds(k + 16, 16)], zero16, mask=full_mask)
        run_batches(1)

    def fix_body(i, carry):
        for u in range(4):
            a = acc[pl.ds((i * 4 + u) * 16, 16)]
            b2 = acc2[pl.ds((i * 4 + u) * 16, 16)]
            mx = jnp.maximum(a, b2)
            acc[pl.ds((i * 4 + u) * 16, 16)] = jnp.where(mx == NEG_INF, 0.0, mx)
        return carry

    lax.fori_loop(0, ACC_ROWS * D // 64, fix_body, 0)

    @pl.when(wid < NW - 1)
    def _():
        pltpu.sync_copy(acc.at[pl.ds(0, NPW * D)],
                        out_ref.at[pl.ds(base * D, NPW * D)])

    @pl.when(wid == NW - 1)
    def _():
        pltpu.sync_copy(acc.at[pl.ds(0, LAST_ROWS * D)],
                        out_ref.at[pl.ds(base * D, LAST_ROWS * D)])


def kernel(x, edge_index, edge_attr, W, b):
    y = _compute_y(x, W, b)
    yflat = y.reshape(NUM_TYPES * N, D)
    src = edge_index[0]
    dst = edge_index[1]
    rowidx = edge_attr * N + src
    outflat = _sc_gather_max(yflat, dst, rowidx)
    return outflat.reshape(N, D)


# speedup vs baseline: 5.7796x; 1.3317x over previous
import functools

import jax
import jax.numpy as jnp
from jax import lax
from jax.experimental import pallas as pl
from jax.experimental.pallas import tpu as pltpu
from jax.experimental.pallas import tpu_sc as plsc

N = 10000
E = 320000
D = 128
NUM_TYPES = 2

NW = 32
NPW = 313
LAST_ROWS = N - (NW - 1) * NPW
ACC_ROWS = NPW + 1
CHUNK = 1280
NCHUNK = E // CHUNK
B = 32
THRESH = 2048
CAP = THRESH + CHUNK + 2 * B
NEG_INF = float("-inf")

BLK = 512
GRID_I = (N + BLK - 1) // BLK


def _matmul_body(x_ref, w_ref, b_ref, y_ref):
    y_ref[0] = (
        jnp.dot(x_ref[...], w_ref[0], preferred_element_type=jnp.float32)
        + b_ref[0]
    )


def _compute_y(x, W, b):
    return pl.pallas_call(
        _matmul_body,
        grid=(NUM_TYPES, GRID_I),
        in_specs=[
            pl.BlockSpec((BLK, D), lambda t, i: (i, 0)),
            pl.BlockSpec((1, D, D), lambda t, i: (t, 0, 0)),
            pl.BlockSpec((1, 1, D), lambda t, i: (t, 0, 0)),
        ],
        out_specs=pl.BlockSpec((1, BLK, D), lambda t, i: (t, i, 0)),
        out_shape=jax.ShapeDtypeStruct((NUM_TYPES, N, D), jnp.float32),
    )(x, W, b.reshape(NUM_TYPES, 1, D))


_MESH = plsc.VectorSubcoreMesh(core_axis_name="c", subcore_axis_name="s")


@functools.partial(
    pl.kernel,
    out_type=jax.ShapeDtypeStruct((N * D,), jnp.float32),
    mesh=_MESH,
    scratch_types=[
        pltpu.VMEM((CHUNK,), jnp.int32),
        pltpu.VMEM((CHUNK,), jnp.int32),
        pltpu.VMEM((CHUNK,), jnp.int32),
        pltpu.VMEM((CHUNK,), jnp.int32),
        pltpu.VMEM((CAP,), jnp.int32),
        pltpu.VMEM((CAP,), jnp.int32),
        pltpu.VMEM((B, D), jnp.float32),
        pltpu.VMEM((B, D), jnp.float32),
        pltpu.VMEM((ACC_ROWS * D,), jnp.float32),
        pltpu.VMEM((ACC_ROWS * D,), jnp.float32),
        pltpu.SemaphoreType.DMA,
        pltpu.SemaphoreType.DMA,
        pltpu.SemaphoreType.DMA,
        pltpu.SemaphoreType.DMA,
        pltpu.SemaphoreType.DMA,
        pltpu.SemaphoreType.DMA,
    ],
    compiler_params=pltpu.CompilerParams(needs_layout_passes=False),
)
def _sc_gather_max(y_ref, dst_ref, row_ref, out_ref,
                   dst0, dst1, row0, row1, pend_off, pend_row,
                   msg0, msg1, acc, acc2, sd0, sd1, sr0, sr1, sg0, sg1):
    c = lax.axis_index("c")
    s = lax.axis_index("s")
    wid = c * 16 + s
    base = wid * NPW
    n_rows = jnp.where(wid == NW - 1, LAST_ROWS, NPW)

    minus_inf = jnp.full((16,), NEG_INF, jnp.float32)
    full_mask = jnp.full((16,), True, jnp.bool_)

    def init_body(i, carry):
        for u in range(4):
            acc[pl.ds((i * 4 + u) * 16, 16)] = minus_inf
            acc2[pl.ds((i * 4 + u) * 16, 16)] = minus_inf
        return carry

    lax.fori_loop(0, ACC_ROWS * D // 64, init_body, 0)

    def issue_chunk(ci, dbuf, rbuf, sd, sr):
        pltpu.async_copy(dst_ref.at[pl.ds(ci * CHUNK, CHUNK)], dbuf, sd)
        pltpu.async_copy(row_ref.at[pl.ds(ci * CHUNK, CHUNK)], rbuf, sr)

    def wait_chunk(dbuf, rbuf, sd, sr):
        pltpu.make_async_copy(dst_ref.at[pl.ds(0, CHUNK)], dbuf, sd).wait()
        pltpu.make_async_copy(row_ref.at[pl.ds(0, CHUNK)], rbuf, sr).wait()

    def gi(p, mref, sref):
        pltpu.async_copy(y_ref.at[pend_row.at[pl.ds(p, B)]], mref, sref)

    def gw(mref, sref):
        pltpu.make_async_copy(y_ref.at[pend_row.at[pl.ds(0, B)]], mref, sref).wait()

    def upd_batch(mref, p):
        def g_body(g, carry):
            off16 = pend_off[pl.ds(p + g * 16, 16)]
            for i in range(16):
                a0 = off16[i] * D
                r = g * 16 + i
                tgt = acc if i % 2 == 0 else acc2
                mvs = [mref[r, pl.ds(j * 16, 16)] for j in range(D // 16)]
                avs = [tgt[pl.ds(a0 + j * 16, 16)] for j in range(D // 16)]
                for j in range(D // 16):
                    tgt[pl.ds(a0 + j * 16, 16)] = jnp.maximum(avs[j], mvs[j])
            return carry

        lax.fori_loop(0, B // 16, g_body, 0)

    def run_batches(nb):
        gi(0, msg0, sg0)

        @pl.when(nb > 1)
        def _():
            gi(B, msg1, sg1)

        def body(t, carry):
            b0 = 2 * t
            gw(msg0, sg0)
            upd_batch(msg0, b0 * B)

            @pl.when(b0 + 2 < nb)
            def _():
                gi((b0 + 2) * B, msg0, sg0)

            @pl.when(b0 + 1 < nb)
            def _():
                gw(msg1, sg1)
                upd_batch(msg1, (b0 + 1) * B)

                @pl.when(b0 + 3 < nb)
                def _():
                    gi((b0 + 3) * B, msg1, sg1)

            return carry

        lax.fori_loop(0, (nb + 1) // 2, body, 0)

    def process_pending(k, thresh):
        def do():
            nb = k // B
            run_batches(nb)
            src = nb * B
            for t in range(B // 16):
                v_off = pend_off[pl.ds(src + t * 16, 16)]
                pend_off[pl.ds(t * 16, 16)] = v_off
                v_row = pend_row[pl.ds(src + t * 16, 16)]
                pend_row[pl.ds(t * 16, 16)] = v_row
            return k - src

        return lax.cond(k >= thresh, do, lambda: k)

    def scan_chunk(dbuf, rbuf, k):
        def scan_body(v, k):
            offs = []
            masks = []
            cnts = []
            for u in range(4):
                d = dbuf[pl.ds((v * 4 + u) * 16, 16)]
                o = d - base
                m = (o >= 0) & (o < n_rows)
                offs.append(o)
                masks.append(m)
                cnts.append(plsc.all_reduce_population_count(m)[0])
            kpos = [k]
            for u in range(3):
                kpos.append(kpos[-1] + cnts[u])
            for u in range(4):
                @pl.when(cnts[u] > 0)
                def _(u=u):
                    plsc.store_compressed(
                        pend_off.at[pl.ds(kpos[u], 16)], offs[u], mask=masks[u])
                    r = rbuf[pl.ds((v * 4 + u) * 16, 16)]
                    plsc.store_compressed(
                        pend_row.at[pl.ds(kpos[u], 16)], r, mask=masks[u])
            return kpos[3] + cnts[3]

        return lax.fori_loop(0, CHUNK // 64, scan_body, k)

    issue_chunk(0, dst0, row0, sd0, sr0)

    def pair_body(t, k):
        c0 = 2 * t
        wait_chunk(dst0, row0, sd0, sr0)
        issue_chunk(c0 + 1, dst1, row1, sd1, sr1)
        k = scan_chunk(dst0, row0, k)
        k = process_pending(k, THRESH)
        wait_chunk(dst1, row1, sd1, sr1)

        @pl.when(c0 + 2 < NCHUNK)
        def _():
            issue_chunk(c0 + 2, dst0, row0, sd0, sr0)

        k = scan_chunk(dst1, row1, k)
        k = process_pending(k, THRESH)
        return k

    k = lax.fori_loop(0, NCHUNK // 2, pair_body, jnp.int32(0))

    k = process_pending(k, B)

    @pl.when(k > 0)
    def _():
        dump = jnp.full((16,), NPW, jnp.int32)
        zero16 = jnp.zeros((16,), jnp.int32)
        plsc.store_compressed(pend_off.at[pl.ds(k, 16)], dump, mask=full_mask)
        plsc.store_compressed(pend_off.at[pl.ds(k + 16, 16)], dump, mask=full_mask)
        plsc.store_compressed(pend_row.at[pl.ds(k, 16)], zero16, mask=full_mask)
        plsc.store_compressed(pend_row.at[pl.ds(k + 16, 16)], zero16, mask=full_mask)
        run_batches(1)

    def fix_body(i, carry):
        for u in range(4):
            a = acc[pl.ds((i * 4 + u) * 16, 16)]
            b2 = acc2[pl.ds((i * 4 + u) * 16, 16)]
            mx = jnp.maximum(a, b2)
            acc[pl.ds((i * 4 + u) * 16, 16)] = jnp.where(mx == NEG_INF, 0.0, mx)
        return carry

    lax.fori_loop(0, ACC_ROWS * D // 64, fix_body, 0)

    @pl.when(wid < NW - 1)
    def _():
        pltpu.sync_copy(acc.at[pl.ds(0, NPW * D)],
                        out_ref.at[pl.ds(base * D, NPW * D)])

    @pl.when(wid == NW - 1)
    def _():
        pltpu.sync_copy(acc.at[pl.ds(0, LAST_ROWS * D)],
                        out_ref.at[pl.ds(base * D, LAST_ROWS * D)])


def kernel(x, edge_index, edge_attr, W, b):
    y = _compute_y(x, W, b)
    yflat = y.reshape(NUM_TYPES * N, D)
    src = edge_index[0]
    dst = edge_index[1]
    rowidx = edge_attr * N + src
    outflat = _sc_gather_max(yflat, dst, rowidx)
    return outflat.reshape(N, D)


# speedup vs baseline: 5.8358x; 1.0097x over previous
import functools

import jax
import jax.numpy as jnp
from jax import lax
from jax.experimental import pallas as pl
from jax.experimental.pallas import tpu as pltpu
from jax.experimental.pallas import tpu_sc as plsc

N = 10000
E = 320000
D = 128
NUM_TYPES = 2

NW = 32
NPW = 313
LAST_ROWS = N - (NW - 1) * NPW
ACC_ROWS = NPW + 1
CHUNK = 1280
NCHUNK = E // CHUNK
B = 32
THRESH = 2048
CAP = THRESH + CHUNK + 2 * B
NEG_INF = float("-inf")

BLK = 512
GRID_I = (N + BLK - 1) // BLK


def _matmul_body(x_ref, w_ref, b_ref, y_ref):
    y_ref[0] = (
        jnp.dot(x_ref[...], w_ref[0], preferred_element_type=jnp.float32)
        + b_ref[0]
    )


def _compute_y(x, W, b):
    return pl.pallas_call(
        _matmul_body,
        grid=(NUM_TYPES, GRID_I),
        in_specs=[
            pl.BlockSpec((BLK, D), lambda t, i: (i, 0)),
            pl.BlockSpec((1, D, D), lambda t, i: (t, 0, 0)),
            pl.BlockSpec((1, 1, D), lambda t, i: (t, 0, 0)),
        ],
        out_specs=pl.BlockSpec((1, BLK, D), lambda t, i: (t, i, 0)),
        out_shape=jax.ShapeDtypeStruct((NUM_TYPES, N, D), jnp.float32),
    )(x, W, b.reshape(NUM_TYPES, 1, D))


_MESH = plsc.VectorSubcoreMesh(core_axis_name="c", subcore_axis_name="s")


@functools.partial(
    pl.kernel,
    out_type=jax.ShapeDtypeStruct((N * D,), jnp.float32),
    mesh=_MESH,
    scratch_types=[
        pltpu.VMEM((CHUNK,), jnp.int32),
        pltpu.VMEM((CHUNK,), jnp.int32),
        pltpu.VMEM((CHUNK,), jnp.int32),
        pltpu.VMEM((CHUNK,), jnp.int32),
        pltpu.VMEM((CAP,), jnp.int32),
        pltpu.VMEM((CAP,), jnp.int32),
        pltpu.VMEM((B, D), jnp.float32),
        pltpu.VMEM((B, D), jnp.float32),
        pltpu.VMEM((ACC_ROWS * D,), jnp.float32),
        pltpu.VMEM((ACC_ROWS * D,), jnp.float32),
        pltpu.SemaphoreType.DMA,
        pltpu.SemaphoreType.DMA,
        pltpu.SemaphoreType.DMA,
        pltpu.SemaphoreType.DMA,
        pltpu.SemaphoreType.DMA,
        pltpu.SemaphoreType.DMA,
    ],
    compiler_params=pltpu.CompilerParams(needs_layout_passes=False),
)
def _sc_gather_max(y_ref, dst_ref, row_ref, out_ref,
                   dst0, dst1, row0, row1, pend_off, pend_row,
                   msg0, msg1, acc, acc2, sd0, sd1, sr0, sr1, sg0, sg1):
    c = lax.axis_index("c")
    s = lax.axis_index("s")
    wid = c * 16 + s
    base = wid * NPW
    n_rows = jnp.where(wid == NW - 1, LAST_ROWS, NPW)
    n_rows_u = n_rows.astype(jnp.uint32)

    minus_inf = jnp.full((16,), NEG_INF, jnp.float32)
    full_mask = jnp.full((16,), True, jnp.bool_)

    def init_body(i, carry):
        for u in range(4):
            acc[pl.ds((i * 4 + u) * 16, 16)] = minus_inf
            acc2[pl.ds((i * 4 + u) * 16, 16)] = minus_inf
        return carry

    lax.fori_loop(0, ACC_ROWS * D // 64, init_body, 0)

    def issue_chunk(ci, dbuf, rbuf, sd, sr):
        pltpu.async_copy(dst_ref.at[pl.ds(ci * CHUNK, CHUNK)], dbuf, sd)
        pltpu.async_copy(row_ref.at[pl.ds(ci * CHUNK, CHUNK)], rbuf, sr)

    def wait_chunk(dbuf, rbuf, sd, sr):
        pltpu.make_async_copy(dst_ref.at[pl.ds(0, CHUNK)], dbuf, sd).wait()
        pltpu.make_async_copy(row_ref.at[pl.ds(0, CHUNK)], rbuf, sr).wait()

    def gi(p, mref, sref):
        pltpu.async_copy(y_ref.at[pend_row.at[pl.ds(p, B)]], mref, sref)

    def gw(mref, sref):
        pltpu.make_async_copy(y_ref.at[pend_row.at[pl.ds(0, B)]], mref, sref).wait()

    def upd_batch(mref, p):
        def g_body(g, carry):
            off16 = pend_off[pl.ds(p + g * 16, 16)]
            for i in range(16):
                a0 = off16[i] * D
                r = g * 16 + i
                tgt = acc if i % 2 == 0 else acc2
                mvs = [mref[r, pl.ds(j * 16, 16)] for j in range(D // 16)]
                avs = [tgt[pl.ds(a0 + j * 16, 16)] for j in range(D // 16)]
                for j in range(D // 16):
                    tgt[pl.ds(a0 + j * 16, 16)] = jnp.maximum(avs[j], mvs[j])
            return carry

        lax.fori_loop(0, B // 16, g_body, 0)

    def run_batches(nb):
        gi(0, msg0, sg0)

        @pl.when(nb > 1)
        def _():
            gi(B, msg1, sg1)

        def body(t, carry):
            b0 = 2 * t
            gw(msg0, sg0)
            upd_batch(msg0, b0 * B)

            @pl.when(b0 + 2 < nb)
            def _():
                gi((b0 + 2) * B, msg0, sg0)

            @pl.when(b0 + 1 < nb)
            def _():
                gw(msg1, sg1)
                upd_batch(msg1, (b0 + 1) * B)

                @pl.when(b0 + 3 < nb)
                def _():
                    gi((b0 + 3) * B, msg1, sg1)

            return carry

        lax.fori_loop(0, (nb + 1) // 2, body, 0)

    def process_pending(k, thresh):
        def do():
            nb = k // B
            run_batches(nb)
            src = nb * B
            for t in range(B // 16):
                v_off = pend_off[pl.ds(src + t * 16, 16)]
                pend_off[pl.ds(t * 16, 16)] = v_off
                v_row = pend_row[pl.ds(src + t * 16, 16)]
                pend_row[pl.ds(t * 16, 16)] = v_row
            return k - src

        return lax.cond(k >= thresh, do, lambda: k)

    SCAN_U = 8

    def scan_chunk(dbuf, rbuf, k):
        def scan_body(v, k):
            offs = []
            masks = []
            cnts = []
            for u in range(SCAN_U):
                d = dbuf[pl.ds((v * SCAN_U + u) * 16, 16)]
                o = d - base
                m = plsc.bitcast(o, jnp.uint32) < n_rows_u
                offs.append(o)
                masks.append(m)
                cnts.append(plsc.all_reduce_population_count(m)[0])
            kpos = [k]
            for u in range(SCAN_U - 1):
                kpos.append(kpos[-1] + cnts[u])
            for u in range(SCAN_U):
                @pl.when(cnts[u] > 0)
                def _(u=u):
                    plsc.store_compressed(
                        pend_off.at[pl.ds(kpos[u], 16)], offs[u], mask=masks[u])
                    r = rbuf[pl.ds((v * SCAN_U + u) * 16, 16)]
                    plsc.store_compressed(
                        pend_row.at[pl.ds(kpos[u], 16)], r, mask=masks[u])
            return kpos[SCAN_U - 1] + cnts[SCAN_U - 1]

        return lax.fori_loop(0, CHUNK // (16 * SCAN_U), scan_body, k)

    issue_chunk(0, dst0, row0, sd0, sr0)

    def pair_body(t, k):
        c0 = 2 * t
        wait_chunk(dst0, row0, sd0, sr0)
        issue_chunk(c0 + 1, dst1, row1, sd1, sr1)
        k = scan_chunk(dst0, row0, k)
        k = process_pending(k, THRESH)
        wait_chunk(dst1, row1, sd1, sr1)

        @pl.when(c0 + 2 < NCHUNK)
        def _():
            issue_chunk(c0 + 2, dst0, row0, sd0, sr0)

        k = scan_chunk(dst1, row1, k)
        k = process_pending(k, THRESH)
        return k

    k = lax.fori_loop(0, NCHUNK // 2, pair_body, jnp.int32(0))

    k = process_pending(k, B)

    @pl.when(k > 0)
    def _():
        dump = jnp.full((16,), NPW, jnp.int32)
        zero16 = jnp.zeros((16,), jnp.int32)
        plsc.store_compressed(pend_off.at[pl.ds(k, 16)], dump, mask=full_mask)
        plsc.store_compressed(pend_off.at[pl.ds(k + 16, 16)], dump, mask=full_mask)
        plsc.store_compressed(pend_row.at[pl.ds(k, 16)], zero16, mask=full_mask)
        plsc.store_compressed(pend_row.at[pl.ds(k + 16, 16)], zero16, mask=full_mask)
        run_batches(1)

    def fix_body(i, carry):
        for u in range(4):
            a = acc[pl.ds((i * 4 + u) * 16, 16)]
            b2 = acc2[pl.ds((i * 4 + u) * 16, 16)]
            mx = jnp.maximum(a, b2)
            acc[pl.ds((i * 4 + u) * 16, 16)] = jnp.where(mx == NEG_INF, 0.0, mx)
        return carry

    lax.fori_loop(0, ACC_ROWS * D // 64, fix_body, 0)

    @pl.when(wid < NW - 1)
    def _():
        pltpu.sync_copy(acc.at[pl.ds(0, NPW * D)],
                        out_ref.at[pl.ds(base * D, NPW * D)])

    @pl.when(wid == NW - 1)
    def _():
        pltpu.sync_copy(acc.at[pl.ds(0, LAST_ROWS * D)],
                        out_ref.at[pl.ds(base * D, LAST_ROWS * D)])


def kernel(x, edge_index, edge_attr, W, b):
    y = _compute_y(x, W, b)
    yflat = y.reshape(NUM_TYPES * N, D)
    src = edge_index[0]
    dst = edge_index[1]
    rowidx = edge_attr * N + src
    outflat = _sc_gather_max(yflat, dst, rowidx)
    return outflat.reshape(N, D)


# speedup vs baseline: 6.2721x; 1.0748x over previous
import functools

import jax
import jax.numpy as jnp
from jax import lax
from jax.experimental import pallas as pl
from jax.experimental.pallas import tpu as pltpu
from jax.experimental.pallas import tpu_sc as plsc

N = 10000
E = 320000
D = 128
NUM_TYPES = 2

NW = 32
NPW = 313
LAST_ROWS = N - (NW - 1) * NPW
ACC_ROWS = NPW + 1
CHUNK = 1280
NCHUNK = E // CHUNK
B = 64
THRESH = 2048
CAP = THRESH + 2 * CHUNK + 2 * B
NEG_INF = float("-inf")

BLK = 512
GRID_I = (N + BLK - 1) // BLK


def _matmul_body(x_ref, w_ref, b_ref, y_ref):
    y_ref[0] = (
        jnp.dot(x_ref[...], w_ref[0], preferred_element_type=jnp.float32)
        + b_ref[0]
    )


def _compute_y(x, W, b):
    return pl.pallas_call(
        _matmul_body,
        grid=(NUM_TYPES, GRID_I),
        in_specs=[
            pl.BlockSpec((BLK, D), lambda t, i: (i, 0)),
            pl.BlockSpec((1, D, D), lambda t, i: (t, 0, 0)),
            pl.BlockSpec((1, 1, D), lambda t, i: (t, 0, 0)),
        ],
        out_specs=pl.BlockSpec((1, BLK, D), lambda t, i: (t, i, 0)),
        out_shape=jax.ShapeDtypeStruct((NUM_TYPES, N, D), jnp.float32),
    )(x, W, b.reshape(NUM_TYPES, 1, D))


_MESH = plsc.VectorSubcoreMesh(core_axis_name="c", subcore_axis_name="s")


@functools.partial(
    pl.kernel,
    out_type=jax.ShapeDtypeStruct((N * D,), jnp.float32),
    mesh=_MESH,
    scratch_types=[
        pltpu.VMEM((CHUNK,), jnp.int32),
        pltpu.VMEM((CHUNK,), jnp.int32),
        pltpu.VMEM((CHUNK,), jnp.int32),
        pltpu.VMEM((CHUNK,), jnp.int32),
        pltpu.VMEM((CAP,), jnp.int32),
        pltpu.VMEM((CAP,), jnp.int32),
        pltpu.VMEM((B, D), jnp.float32),
        pltpu.VMEM((B, D), jnp.float32),
        pltpu.VMEM((B, D), jnp.float32),
        pltpu.VMEM((ACC_ROWS * D,), jnp.float32),
        pltpu.VMEM((ACC_ROWS * D,), jnp.float32),
        pltpu.SemaphoreType.DMA,
        pltpu.SemaphoreType.DMA,
        pltpu.SemaphoreType.DMA,
        pltpu.SemaphoreType.DMA,
        pltpu.SemaphoreType.DMA,
        pltpu.SemaphoreType.DMA,
        pltpu.SemaphoreType.DMA,
    ],
    compiler_params=pltpu.CompilerParams(needs_layout_passes=False),
)
def _sc_gather_max(y_ref, dst_ref, row_ref, out_ref,
                   dst0, dst1, row0, row1, pend_off, pend_row,
                   msg0, msg1, msg2, acc, acc2, sd0, sd1, sr0, sr1,
                   sg0, sg1, sg2):
    c = lax.axis_index("c")
    s = lax.axis_index("s")
    wid = c * 16 + s
    base = wid * NPW
    n_rows = jnp.where(wid == NW - 1, LAST_ROWS, NPW)
    n_rows_u = n_rows.astype(jnp.uint32)

    minus_inf = jnp.full((16,), NEG_INF, jnp.float32)
    full_mask = jnp.full((16,), True, jnp.bool_)

    def init_body(i, carry):
        for u in range(4):
            acc[pl.ds((i * 4 + u) * 16, 16)] = minus_inf
            acc2[pl.ds((i * 4 + u) * 16, 16)] = minus_inf
        return carry

    lax.fori_loop(0, ACC_ROWS * D // 64, init_body, 0)

    def issue_chunk(ci, dbuf, rbuf, sd, sr):
        pltpu.async_copy(dst_ref.at[pl.ds(ci * CHUNK, CHUNK)], dbuf, sd)
        pltpu.async_copy(row_ref.at[pl.ds(ci * CHUNK, CHUNK)], rbuf, sr)

    def wait_chunk(dbuf, rbuf, sd, sr):
        pltpu.make_async_copy(dst_ref.at[pl.ds(0, CHUNK)], dbuf, sd).wait()
        pltpu.make_async_copy(row_ref.at[pl.ds(0, CHUNK)], rbuf, sr).wait()

    def gi(p, mref, sref):
        pltpu.async_copy(y_ref.at[pend_row.at[pl.ds(p, B)]], mref, sref)

    def gw(mref, sref):
        pltpu.make_async_copy(y_ref.at[pend_row.at[pl.ds(0, B)]], mref, sref).wait()

    def upd_batch(mref, p):
        def g_body(g, carry):
            off16 = pend_off[pl.ds(p + g * 16, 16)]
            a0s = [off16[i] * D for i in range(16)]
            for i2 in range(8):
                iA = 2 * i2
                iB = iA + 1
                rA = g * 16 + iA
                rB = g * 16 + iB
                nj = D // 16
                mvA = [mref[rA, pl.ds(j * 16, 16)] for j in range(nj)]
                avA = [acc[pl.ds(a0s[iA] + j * 16, 16)] for j in range(nj)]
                mvB = [mref[rB, pl.ds(j * 16, 16)] for j in range(nj)]
                avB = [acc2[pl.ds(a0s[iB] + j * 16, 16)] for j in range(nj)]
                for j in range(nj):
                    acc[pl.ds(a0s[iA] + j * 16, 16)] = jnp.maximum(avA[j], mvA[j])
                for j in range(nj):
                    acc2[pl.ds(a0s[iB] + j * 16, 16)] = jnp.maximum(avB[j], mvB[j])
            return carry

        lax.fori_loop(0, B // 16, g_body, 0)

    SLOTS = ((msg0, sg0), (msg1, sg1), (msg2, sg2))
    ND = len(SLOTS)

    def run_batches(nb):
        gi(0, msg0, sg0)

        @pl.when(nb > 1)
        def _():
            gi(B, msg1, sg1)

        @pl.when(nb > 2)
        def _():
            gi(2 * B, msg2, sg2)

        def body(t, carry):
            b0 = ND * t

            def step(q):
                bq = b0 + q
                mref, sref = SLOTS[q]

                @pl.when(bq < nb)
                def _():
                    gw(mref, sref)
                    upd_batch(mref, bq * B)

                    @pl.when(bq + ND < nb)
                    def _():
                        gi((bq + ND) * B, mref, sref)

            for q in range(ND):
                step(q)
            return carry

        lax.fori_loop(0, (nb + ND - 1) // ND, body, 0)

    def process_pending(k, thresh):
        def do():
            nb = k // B
            run_batches(nb)
            src = nb * B
            for t in range(B // 16):
                v_off = pend_off[pl.ds(src + t * 16, 16)]
                pend_off[pl.ds(t * 16, 16)] = v_off
                v_row = pend_row[pl.ds(src + t * 16, 16)]
                pend_row[pl.ds(t * 16, 16)] = v_row
            return k - src

        return lax.cond(k >= thresh, do, lambda: k)

    SCAN_U = 8

    def scan_chunk(dbuf, rbuf, k):
        def scan_body(v, k):
            offs = []
            masks = []
            cnts = []
            for u in range(SCAN_U):
                d = dbuf[pl.ds((v * SCAN_U + u) * 16, 16)]
                o = d - base
                m = plsc.bitcast(o, jnp.uint32) < n_rows_u
                offs.append(o)
                masks.append(m)
                cnts.append(plsc.all_reduce_population_count(m)[0])
            kpos = [k]
            for u in range(SCAN_U - 1):
                kpos.append(kpos[-1] + cnts[u])
            for u in range(SCAN_U):
                @pl.when(cnts[u] > 0)
                def _(u=u):
                    plsc.store_compressed(
                        pend_off.at[pl.ds(kpos[u], 16)], offs[u], mask=masks[u])
                    r = rbuf[pl.ds((v * SCAN_U + u) * 16, 16)]
                    plsc.store_compressed(
                        pend_row.at[pl.ds(kpos[u], 16)], r, mask=masks[u])
            return kpos[SCAN_U - 1] + cnts[SCAN_U - 1]

        return lax.fori_loop(0, CHUNK // (16 * SCAN_U), scan_body, k)

    issue_chunk(0, dst0, row0, sd0, sr0)

    def pair_body(t, k):
        c0 = 2 * t
        wait_chunk(dst0, row0, sd0, sr0)
        issue_chunk(c0 + 1, dst1, row1, sd1, sr1)
        k = scan_chunk(dst0, row0, k)
        wait_chunk(dst1, row1, sd1, sr1)

        @pl.when(c0 + 2 < NCHUNK)
        def _():
            issue_chunk(c0 + 2, dst0, row0, sd0, sr0)

        k = scan_chunk(dst1, row1, k)
        k = process_pending(k, THRESH)
        return k

    k = lax.fori_loop(0, NCHUNK // 2, pair_body, jnp.int32(0))

    k = process_pending(k, B)

    @pl.when(k > 0)
    def _():
        dump = jnp.full((16,), NPW, jnp.int32)
        zero16 = jnp.zeros((16,), jnp.int32)
        for t in range(B // 16):
            plsc.store_compressed(pend_off.at[pl.ds(k + 16 * t, 16)], dump,
                                  mask=full_mask)
            plsc.store_compressed(pend_row.at[pl.ds(k + 16 * t, 16)], zero16,
                                  mask=full_mask)
        gi(0, msg0, sg0)
        gw(msg0, sg0)
        upd_batch(msg0, 0)

    def fix_body(i, carry):
        for u in range(4):
            a = acc[pl.ds((i * 4 + u) * 16, 16)]
            b2 = acc2[pl.ds((i * 4 + u) * 16, 16)]
            mx = jnp.maximum(a, b2)
            acc[pl.ds((i * 4 + u) * 16, 16)] = jnp.where(mx == NEG_INF, 0.0, mx)
        return carry

    lax.fori_loop(0, ACC_ROWS * D // 64, fix_body, 0)

    @pl.when(wid < NW - 1)
    def _():
        pltpu.sync_copy(acc.at[pl.ds(0, NPW * D)],
                        out_ref.at[pl.ds(base * D, NPW * D)])

    @pl.when(wid == NW - 1)
    def _():
        pltpu.sync_copy(acc.at[pl.ds(0, LAST_ROWS * D)],
                        out_ref.at[pl.ds(base * D, LAST_ROWS * D)])


def kernel(x, edge_index, edge_attr, W, b):
    y = _compute_y(x, W, b)
    yflat = y.reshape(NUM_TYPES * N, D)
    src = edge_index[0]
    dst = edge_index[1]
    rowidx = edge_attr * N + src
    outflat = _sc_gather_max(yflat, dst, rowidx)
    return outflat.reshape(N, D)
